# baseline (speedup 1.0000x reference)
import functools

import jax
import jax.numpy as jnp
from jax import lax
from jax.experimental import pallas as pl
from jax.experimental.pallas import tpu as pltpu
from jax.experimental.pallas import tpu_sc as plsc

_ALPHA = 0.2
_BETA = 0.2
_GAMMA = 0.2
_DELTA = 1.0
_EPS = 1e-6


def _stage1_kernel(logits_ref, tcol_ref, st_ref, scol_ref, hist_ref):
    x = logits_ref[...]
    m = jnp.max(x, axis=1, keepdims=True)
    e = jnp.exp(x - m)
    z = jnp.sum(e, axis=1, keepdims=True)
    sc = e / z
    st_ref[...] = jnp.transpose(sc)
    rb, cp = x.shape
    iota_c = lax.broadcasted_iota(jnp.int32, (rb, cp), 1)
    oh = (iota_c == tcol_ref[...]).astype(jnp.float32)
    scol_ref[...] = jnp.sum(sc * oh, axis=1, keepdims=True)
    hpart = jnp.sum(oh, axis=0, keepdims=True)

    @pl.when(pl.program_id(0) == 0)
    def _init():
        hist_ref[...] = hpart

    @pl.when(pl.program_id(0) != 0)
    def _acc():
        hist_ref[...] = hist_ref[...] + hpart


def _sc_count_kernel(st_hbm, s_hbm, t_hbm, out_hbm, mys_ref, myt_ref,
                     rows_ref, myq_ref, acc_ref, sem, *, batch, per, rowlen):
    nc = 2
    wid = lax.axis_index("s") * nc + lax.axis_index("c")
    base = wid * per
    pltpu.sync_copy(s_hbm.at[pl.ds(base, per)], mys_ref)
    pltpu.sync_copy(t_hbm.at[pl.ds(base, per)], myt_ref)
    iota16 = lax.iota(jnp.int32, 16)
    zeros16 = jnp.zeros((16,), jnp.int32)
    ones16 = jnp.ones((16,), jnp.float32)
    zf16 = jnp.zeros((16,), jnp.float32)

    dn0 = lax.GatherDimensionNumbers(offset_dims=(),
                                     collapsed_slice_dims=(0,),
                                     start_index_map=(0,))

    def gbody(g, _):
        idxv = myt_ref[pl.ds(g * 16, 16)]
        pltpu.async_copy(st_hbm.at[idxv], rows_ref, sem).wait()
        schunk = mys_ref[pl.ds(g * 16, 16)]

        def sbody(r, _s):
            m = g * 16 + r
            s_splat = lax.gather(schunk, (zeros16 + r).reshape(16, 1), dn0,
                                 (1,),
                                 mode=lax.GatherScatterMode.PROMISE_IN_BOUNDS)
            gi = base + m
            acc_ref[...] = zf16

            def cbody(ch, _c):
                v = rows_ref[r, pl.ds(ch * 16, 16)]
                jv = iota16 + ch * 16
                gt = v > s_splat
                tie = (v == s_splat) & (jv < gi)
                acc_ref[...] = acc_ref[...] + jnp.where(gt | tie, ones16,
                                                        zf16)
                return 0

            lax.fori_loop(0, rowlen // 16, cbody, 0)
            myq_ref[pl.ds(m * 16, 16)] = acc_ref[...]
            return 0

        lax.fori_loop(0, 16, sbody, 0)
        return 0

    lax.fori_loop(0, per // 16, gbody, 0)
    pltpu.sync_copy(myq_ref, out_hbm.at[pl.ds(base * 16, per * 16)])


def _stage2_kernel(qpart_ref, ticol_ref, scol_ref, srow_ref, trow_ref,
                   hist_ref, acc_ref, *, blk, batch, num_classes):
    i = pl.program_id(0)
    i0 = i * blk
    q = jnp.sum(qpart_ref[...], axis=1, keepdims=True)
    s_i = scol_ref[...]
    t_i = ticol_ref[...]
    s_all = srow_ref[...]
    t_all = trow_ref[...]
    tmatch = t_i == t_all
    jv = lax.broadcasted_iota(jnp.int32, (blk, batch), 1)
    iv = i0 + lax.broadcasted_iota(jnp.int32, (blk, batch), 0)
    before = jv < iv
    mk = tmatch & ((s_all > s_i) | ((s_all == s_i) & before))
    k = jnp.sum(mk.astype(jnp.float32), axis=1, keepdims=True)
    p_i = jnp.sum(tmatch.astype(jnp.float32), axis=1, keepdims=True)
    bf = jnp.float32(batch)

    def g_of(x, p):
        return jnp.exp(_GAMMA * jnp.log(1.0 - x / (p + _EPS)))

    gk = g_of(k, p_i)
    gk1 = g_of(k + 1.0, p_i)
    t_q = q * (q + 1.0) * 0.5
    contrib = (q * (gk1 * (k + 1.0) - gk * k)
               + t_q * (gk - gk1)) / (bf - p_i + _EPS)
    focal = jnp.sum((1.0 - s_i) ** 2 * jnp.log(s_i))
    val = jnp.reshape(
        ((1.0 - _ALPHA) * _BETA * jnp.sum(contrib)
         + _ALPHA * _DELTA * focal) / bf, (1, 1))

    @pl.when(i == 0)
    def _init():
        hist = hist_ref[...]
        cp = hist.shape[1]
        mask = (lax.broadcasted_iota(jnp.int32, (1, cp), 1) < num_classes)
        gp = jnp.exp(_GAMMA * jnp.log(_EPS / (hist + _EPS)))
        tb = bf * (bf + 1.0) * 0.5
        ct = jnp.where(mask, gp * (tb - hist * bf) / (bf - hist + _EPS), 0.0)
        acc_ref[...] = val + jnp.reshape(
            jnp.sum(ct) * ((1.0 - _ALPHA) * _BETA / bf), (1, 1))

    @pl.when(i != 0)
    def _acc():
        acc_ref[...] = acc_ref[...] + val


@jax.jit
def kernel(logits, targets):
    b, c = logits.shape
    cp = 1024
    rb = 512
    blk = 512
    nworkers = 32
    per = b // nworkers
    logits_p = jnp.pad(logits, ((0, 0), (0, cp - c)),
                       constant_values=-1e30)
    t32 = targets.astype(jnp.int32)
    t_col = t32.reshape(b, 1)
    t_row = t32.reshape(1, b)

    scores_t, s_col, hist = pl.pallas_call(
        _stage1_kernel,
        grid=(b // rb,),
        in_specs=[
            pl.BlockSpec((rb, cp), lambda i: (i, 0)),
            pl.BlockSpec((rb, 1), lambda i: (i, 0)),
        ],
        out_specs=[
            pl.BlockSpec((cp, rb), lambda i: (0, i)),
            pl.BlockSpec((rb, 1), lambda i: (i, 0)),
            pl.BlockSpec((1, cp), lambda i: (0, 0)),
        ],
        out_shape=[
            jax.ShapeDtypeStruct((cp, b), jnp.float32),
            jax.ShapeDtypeStruct((b, 1), jnp.float32),
            jax.ShapeDtypeStruct((1, cp), jnp.float32),
        ],
    )(logits_p, t_col)

    s_flat = s_col.reshape(b)
    s_row = s_col.reshape(1, b)

    mesh = plsc.VectorSubcoreMesh(core_axis_name="c", subcore_axis_name="s")
    qpart = pl.kernel(
        functools.partial(_sc_count_kernel, batch=b, per=per, rowlen=b),
        mesh=mesh,
        out_type=jax.ShapeDtypeStruct((b * 16,), jnp.float32),
        scratch_types=[
            pltpu.VMEM((per,), jnp.float32),
            pltpu.VMEM((per,), jnp.int32),
            pltpu.VMEM((16, b), jnp.float32),
            pltpu.VMEM((per * 16,), jnp.float32),
            pltpu.VMEM((16,), jnp.float32),
            pltpu.SemaphoreType.DMA,
        ],
    )(scores_t, s_flat, t32)
    qpart = qpart.reshape(b, 16)

    acc = pl.pallas_call(
        functools.partial(_stage2_kernel, blk=blk, batch=b, num_classes=c),
        grid=(b // blk,),
        in_specs=[
            pl.BlockSpec((blk, 16), lambda i: (i, 0)),
            pl.BlockSpec((blk, 1), lambda i: (i, 0)),
            pl.BlockSpec((blk, 1), lambda i: (i, 0)),
            pl.BlockSpec((1, b), lambda i: (0, 0)),
            pl.BlockSpec((1, b), lambda i: (0, 0)),
            pl.BlockSpec((1, cp), lambda i: (0, 0)),
        ],
        out_specs=pl.BlockSpec((1, 1), lambda i: (0, 0)),
        out_shape=jax.ShapeDtypeStruct((1, 1), jnp.float32),
    )(qpart, t_col, s_col, s_row, t_row, hist)

    return acc[0, 0]


# speedup vs baseline: 2.3125x; 2.3125x over previous
import functools

import jax
import jax.numpy as jnp
from jax import lax
from jax.experimental import pallas as pl
from jax.experimental.pallas import tpu as pltpu
from jax.experimental.pallas import tpu_sc as plsc

_ALPHA = 0.2
_BETA = 0.2
_GAMMA = 0.2
_DELTA = 1.0
_EPS = 1e-6


def _stage1_kernel(logits_ref, tcol_ref, hi_ref, mid_ref, st_ref,
                   scol_ref, hist_ref):
    x = logits_ref[...]
    m = jnp.max(x, axis=1, keepdims=True)
    e = jnp.exp(x - m)
    z = jnp.sum(e, axis=1, keepdims=True)
    sc = e / z
    hi = sc.astype(jnp.bfloat16)
    r = sc - hi.astype(jnp.float32)
    hi_ref[...] = hi
    mid_ref[...] = r.astype(jnp.bfloat16)
    st_ref[...] = jnp.transpose(sc)
    rb, cp = x.shape
    iota_c = lax.broadcasted_iota(jnp.int32, (rb, cp), 1)
    oh = (iota_c == tcol_ref[...]).astype(jnp.float32)
    scol_ref[...] = jnp.sum(sc * oh, axis=1, keepdims=True)
    hpart = jnp.sum(oh, axis=0, keepdims=True)

    @pl.when(pl.program_id(0) == 0)
    def _init():
        hist_ref[...] = hpart

    @pl.when(pl.program_id(0) != 0)
    def _acc():
        hist_ref[...] = hist_ref[...] + hpart


def _sc_count_kernel(st_hbm, s_hbm, t_hbm, out_hbm, mys_ref, myt_ref,
                     rows0_ref, rows1_ref, myq_ref, acc_ref, sem0, sem1, *,
                     batch, per, unroll):
    nc = 2
    wid = lax.axis_index("s") * nc + lax.axis_index("c")
    base = wid * per
    pltpu.sync_copy(s_hbm.at[pl.ds(base, per)], mys_ref)
    pltpu.sync_copy(t_hbm.at[pl.ds(base, per)], myt_ref)
    iota16 = lax.iota(jnp.int32, 16)
    zeros16 = jnp.zeros((16,), jnp.int32)
    ones16 = jnp.ones((16,), jnp.float32)
    zf16 = jnp.zeros((16,), jnp.float32)
    dn0 = lax.GatherDimensionNumbers(offset_dims=(),
                                     collapsed_slice_dims=(0,),
                                     start_index_map=(0,))
    grp = 8
    ngrp = per // grp
    npairs = ngrp // 2

    def _issue(g, rows_ref, sem):
        return pltpu.async_copy(st_hbm.at[myt_ref.at[pl.ds(g * grp, grp)]],
                                rows_ref, sem)

    def _process(rows_ref, g):
        schunk = mys_ref[pl.ds((g // 2) * 16, 16)]
        lbase = (g % 2) * grp

        def sbody(r, _s):
            m = g * grp + r
            s_splat = lax.gather(schunk, (zeros16 + (lbase + r)).reshape(
                16, 1), dn0, (1,),
                mode=lax.GatherScatterMode.PROMISE_IN_BOUNDS)
            gi = base + m
            acc_ref[...] = zf16

            def cbody(cc, _c):
                tot = zf16
                for u in range(unroll):
                    ch = cc * unroll + u
                    v = rows_ref[r, pl.ds(ch * 16, 16)]
                    jv = iota16 + ch * 16
                    gt = v > s_splat
                    tie = (v == s_splat) & (jv < gi)
                    tot = tot + jnp.where(gt | tie, ones16, zf16)
                acc_ref[...] = acc_ref[...] + tot
                return 0

            lax.fori_loop(0, batch // (16 * unroll), cbody, 0)
            myq_ref[pl.ds(m * 16, 16)] = acc_ref[...]
            return 0

        lax.fori_loop(0, grp, sbody, 0)

    _issue(0, rows0_ref, sem0)

    def pbody(p, _):
        pltpu.make_async_copy(st_hbm.at[myt_ref.at[pl.ds(0, grp)]],
                              rows0_ref, sem0).wait()
        _issue(2 * p + 1, rows1_ref, sem1)
        _process(rows0_ref, 2 * p)

        @pl.when(p < npairs - 1)
        def _next():
            _issue(2 * p + 2, rows0_ref, sem0)

        pltpu.make_async_copy(st_hbm.at[myt_ref.at[pl.ds(0, grp)]],
                              rows1_ref, sem1).wait()
        _process(rows1_ref, 2 * p + 1)
        return 0

    lax.fori_loop(0, npairs, pbody, 0)
    pltpu.sync_copy(myq_ref, out_hbm.at[pl.ds(base * 16, per * 16)])


def _tc_count_kernel(hi_ref, mid_ref, ticol_ref, srow_ref, qout_ref,
                     q_ref, oh_ref, *, blk, nblk, ioff):
    j = pl.program_id(0)
    i = pl.program_id(1)
    ig = i + ioff
    cp = hi_ref.shape[1]
    isl = pl.ds(i * blk, blk)

    @pl.when(j == 0)
    def _build_oh():
        oh_ref[isl, :] = (lax.broadcasted_iota(jnp.int32, (blk, cp), 1)
                          == ticol_ref[...]).astype(jnp.bfloat16)

    ohb = oh_ref[isl, :]
    dn = (((1,), (1,)), ((), ()))
    g_mat = (lax.dot_general(hi_ref[...], ohb, dn,
                             preferred_element_type=jnp.float32)
             + lax.dot_general(mid_ref[...], ohb, dn,
                               preferred_element_type=jnp.float32))
    s_i = srow_ref[...]

    def _store(q_part):
        qp = jnp.sum(q_part.astype(jnp.float32), axis=0, keepdims=True)

        @pl.when(j == 0)
        def _():
            q_ref[:, isl] = qp

        @pl.when(j != 0)
        def _():
            q_ref[:, isl] = q_ref[:, isl] + qp

    @pl.when(j < ig)
    def _before():
        _store(g_mat >= s_i)

    @pl.when(j > ig)
    def _after():
        _store(g_mat > s_i)

    @pl.when(j == ig)
    def _diag():
        jio = lax.broadcasted_iota(jnp.int32, (blk, blk), 0)
        iio = lax.broadcasted_iota(jnp.int32, (blk, blk), 1)
        _store(((g_mat > s_i) | ((g_mat == s_i) & (jio < iio)))
               & (jio != iio))

    @pl.when(j == nblk - 1)
    def _finish():
        qout_ref[...] = jnp.transpose(q_ref[:, isl])


def _final_kernel(qpart_ref, qtc_ref, ticol_ref, scol_ref, srow_ref,
                  trow_ref, hist_ref, acc_ref, *, blk, batch, num_classes,
                  nsc_blocks):
    i = pl.program_id(0)
    i0 = i * blk
    q_sc = jnp.sum(qpart_ref[...], axis=1, keepdims=True)
    q = jnp.where(i < nsc_blocks, q_sc, qtc_ref[...])
    s_i = scol_ref[...]
    t_i = ticol_ref[...]
    s_all = srow_ref[...]
    t_all = trow_ref[...]
    tmatch = t_i == t_all
    jv = lax.broadcasted_iota(jnp.int32, (blk, batch), 1)
    iv = i0 + lax.broadcasted_iota(jnp.int32, (blk, batch), 0)
    before = jv < iv
    mk = tmatch & ((s_all > s_i) | ((s_all == s_i) & before))
    k = jnp.sum(mk.astype(jnp.float32), axis=1, keepdims=True)
    p_i = jnp.sum(tmatch.astype(jnp.float32), axis=1, keepdims=True)
    bf = jnp.float32(batch)

    def g_of(x, p):
        return jnp.exp(_GAMMA * jnp.log(1.0 - x / (p + _EPS)))

    gk = g_of(k, p_i)
    gk1 = g_of(k + 1.0, p_i)
    t_q = q * (q + 1.0) * 0.5
    contrib = (q * (gk1 * (k + 1.0) - gk * k)
               + t_q * (gk - gk1)) / (bf - p_i + _EPS)
    focal = jnp.sum((1.0 - s_i) ** 2 * jnp.log(s_i))
    val = jnp.reshape(
        ((1.0 - _ALPHA) * _BETA * jnp.sum(contrib)
         + _ALPHA * _DELTA * focal) / bf, (1, 1))

    @pl.when(i == 0)
    def _init():
        hist = hist_ref[...]
        cp = hist.shape[1]
        mask = (lax.broadcasted_iota(jnp.int32, (1, cp), 1) < num_classes)
        gp = jnp.exp(_GAMMA * jnp.log(_EPS / (hist + _EPS)))
        tb = bf * (bf + 1.0) * 0.5
        ct = jnp.where(mask, gp * (tb - hist * bf) / (bf - hist + _EPS), 0.0)
        acc_ref[...] = val + jnp.reshape(
            jnp.sum(ct) * ((1.0 - _ALPHA) * _BETA / bf), (1, 1))

    @pl.when(i != 0)
    def _acc():
        acc_ref[...] = acc_ref[...] + val


@jax.jit
def kernel(logits, targets):
    b, c = logits.shape
    cp = 1024
    rb = 512
    blk = 1024
    fblk = 512
    sc_samples = 2048
    nworkers = 32
    per = sc_samples // nworkers
    ioff = sc_samples // blk
    nblk = b // blk
    logits_p = jnp.pad(logits, ((0, 0), (0, cp - c)),
                       constant_values=-1e30)
    t32 = targets.astype(jnp.int32)
    t_col = t32.reshape(b, 1)
    t_row = t32.reshape(1, b)

    s_hi, s_mid, scores_t, s_col, hist = pl.pallas_call(
        _stage1_kernel,
        grid=(b // rb,),
        in_specs=[
            pl.BlockSpec((rb, cp), lambda i: (i, 0)),
            pl.BlockSpec((rb, 1), lambda i: (i, 0)),
        ],
        out_specs=[
            pl.BlockSpec((rb, cp), lambda i: (i, 0)),
            pl.BlockSpec((rb, cp), lambda i: (i, 0)),
            pl.BlockSpec((cp, rb), lambda i: (0, i)),
            pl.BlockSpec((rb, 1), lambda i: (i, 0)),
            pl.BlockSpec((1, cp), lambda i: (0, 0)),
        ],
        out_shape=[
            jax.ShapeDtypeStruct((b, cp), jnp.bfloat16),
            jax.ShapeDtypeStruct((b, cp), jnp.bfloat16),
            jax.ShapeDtypeStruct((cp, b), jnp.float32),
            jax.ShapeDtypeStruct((b, 1), jnp.float32),
            jax.ShapeDtypeStruct((1, cp), jnp.float32),
        ],
    )(logits_p, t_col)

    s_flat = s_col.reshape(b)
    s_row = s_col.reshape(1, b)

    mesh = plsc.VectorSubcoreMesh(core_axis_name="c", subcore_axis_name="s")
    qpart = pl.kernel(
        functools.partial(_sc_count_kernel, batch=b, per=per, unroll=8),
        mesh=mesh,
        out_type=jax.ShapeDtypeStruct((sc_samples * 16,), jnp.float32),
        scratch_types=[
            pltpu.VMEM((per,), jnp.float32),
            pltpu.VMEM((per,), jnp.int32),
            pltpu.VMEM((8, b), jnp.float32),
            pltpu.VMEM((8, b), jnp.float32),
            pltpu.VMEM((per * 16,), jnp.float32),
            pltpu.VMEM((16,), jnp.float32),
            pltpu.SemaphoreType.DMA,
            pltpu.SemaphoreType.DMA,
        ],
    )(scores_t, s_flat, t32)
    qpart = jnp.pad(qpart.reshape(sc_samples, 16),
                    ((0, b - sc_samples), (0, 0)))

    q_tc = pl.pallas_call(
        functools.partial(_tc_count_kernel, blk=blk, nblk=nblk, ioff=ioff),
        grid=(nblk, (b - sc_samples) // blk),
        in_specs=[
            pl.BlockSpec((blk, cp), lambda j, i: (j, 0)),
            pl.BlockSpec((blk, cp), lambda j, i: (j, 0)),
            pl.BlockSpec((blk, 1), lambda j, i: (i + 2, 0)),
            pl.BlockSpec((1, blk), lambda j, i: (0, i + 2)),
        ],
        out_specs=pl.BlockSpec((blk, 1), lambda j, i: (i + 2, 0)),
        out_shape=jax.ShapeDtypeStruct((b, 1), jnp.float32),
        scratch_shapes=[
            pltpu.VMEM((1, b - sc_samples), jnp.float32),
            pltpu.VMEM((b - sc_samples, cp), jnp.bfloat16),
        ],
    )(s_hi, s_mid, t_col, s_row)

    acc = pl.pallas_call(
        functools.partial(_final_kernel, blk=fblk, batch=b, num_classes=c,
                          nsc_blocks=sc_samples // fblk),
        grid=(b // fblk,),
        in_specs=[
            pl.BlockSpec((fblk, 16), lambda i: (i, 0)),
            pl.BlockSpec((fblk, 1), lambda i: (i, 0)),
            pl.BlockSpec((fblk, 1), lambda i: (i, 0)),
            pl.BlockSpec((fblk, 1), lambda i: (i, 0)),
            pl.BlockSpec((1, b), lambda i: (0, 0)),
            pl.BlockSpec((1, b), lambda i: (0, 0)),
            pl.BlockSpec((1, cp), lambda i: (0, 0)),
        ],
        out_specs=pl.BlockSpec((1, 1), lambda i: (0, 0)),
        out_shape=jax.ShapeDtypeStruct((1, 1), jnp.float32),
    )(qpart, q_tc, t_col, s_col, s_row, t_row, hist)

    return acc[0, 0]


# speedup vs baseline: 2.3136x; 1.0005x over previous
import functools

import jax
import jax.numpy as jnp
from jax import lax
from jax.experimental import pallas as pl
from jax.experimental.pallas import tpu as pltpu
from jax.experimental.pallas import tpu_sc as plsc

_ALPHA = 0.2
_BETA = 0.2
_GAMMA = 0.2
_DELTA = 1.0
_EPS = 1e-6


def _stage1_kernel(logits_ref, tcol_ref, hi_ref, mid_ref, st_ref,
                   scol_ref, hist_ref):
    x = logits_ref[...]
    m = jnp.max(x, axis=1, keepdims=True)
    e = jnp.exp(x - m)
    z = jnp.sum(e, axis=1, keepdims=True)
    sc = e / z
    hi = sc.astype(jnp.bfloat16)
    r = sc - hi.astype(jnp.float32)
    hi_ref[...] = hi
    mid_ref[...] = r.astype(jnp.bfloat16)
    st_ref[...] = jnp.transpose(sc)
    rb, cp = x.shape
    iota_c = lax.broadcasted_iota(jnp.int32, (rb, cp), 1)
    oh = (iota_c == tcol_ref[...]).astype(jnp.float32)
    scol_ref[...] = jnp.sum(sc * oh, axis=1, keepdims=True)
    hpart = jnp.sum(oh, axis=0, keepdims=True)

    @pl.when(pl.program_id(0) == 0)
    def _init():
        hist_ref[...] = hpart

    @pl.when(pl.program_id(0) != 0)
    def _acc():
        hist_ref[...] = hist_ref[...] + hpart


def _sc_count_kernel(st_hbm, s_hbm, t_hbm, out_hbm, mys_ref, myt_ref,
                     rows0_ref, rows1_ref, myq_ref, acc_ref, sem0, sem1, *,
                     batch, per, unroll):
    nc = 2
    wid = lax.axis_index("s") * nc + lax.axis_index("c")
    base = wid * per
    pltpu.sync_copy(s_hbm.at[pl.ds(base, per)], mys_ref)
    pltpu.sync_copy(t_hbm.at[pl.ds(base, per)], myt_ref)
    iota16 = lax.iota(jnp.int32, 16)
    zeros16 = jnp.zeros((16,), jnp.int32)
    ones16 = jnp.ones((16,), jnp.float32)
    zf16 = jnp.zeros((16,), jnp.float32)
    dn0 = lax.GatherDimensionNumbers(offset_dims=(),
                                     collapsed_slice_dims=(0,),
                                     start_index_map=(0,))
    grp = 8
    ngrp = per // grp
    npairs = ngrp // 2

    def _issue(g, rows_ref, sem):
        return pltpu.async_copy(st_hbm.at[myt_ref.at[pl.ds(g * grp, grp)]],
                                rows_ref, sem)

    def _process(rows_ref, g):
        schunk = mys_ref[pl.ds((g // 2) * 16, 16)]
        lbase = (g % 2) * grp

        def sbody(r, _s):
            m = g * grp + r
            s_splat = lax.gather(schunk, (zeros16 + (lbase + r)).reshape(
                16, 1), dn0, (1,),
                mode=lax.GatherScatterMode.PROMISE_IN_BOUNDS)
            gi = base + m
            acc_ref[...] = zf16

            def cbody(cc, _c):
                tot = zf16
                for u in range(unroll):
                    ch = cc * unroll + u
                    v = rows_ref[r, pl.ds(ch * 16, 16)]
                    jv = iota16 + ch * 16
                    gt = v > s_splat
                    tie = (v == s_splat) & (jv < gi)
                    tot = tot + jnp.where(gt | tie, ones16, zf16)
                acc_ref[...] = acc_ref[...] + tot
                return 0

            lax.fori_loop(0, batch // (16 * unroll), cbody, 0)
            myq_ref[pl.ds(m * 16, 16)] = acc_ref[...]
            return 0

        lax.fori_loop(0, grp, sbody, 0)

    _issue(0, rows0_ref, sem0)

    def pbody(p, _):
        pltpu.make_async_copy(st_hbm.at[myt_ref.at[pl.ds(0, grp)]],
                              rows0_ref, sem0).wait()
        _issue(2 * p + 1, rows1_ref, sem1)
        _process(rows0_ref, 2 * p)

        @pl.when(p < npairs - 1)
        def _next():
            _issue(2 * p + 2, rows0_ref, sem0)

        pltpu.make_async_copy(st_hbm.at[myt_ref.at[pl.ds(0, grp)]],
                              rows1_ref, sem1).wait()
        _process(rows1_ref, 2 * p + 1)
        return 0

    lax.fori_loop(0, npairs, pbody, 0)
    pltpu.sync_copy(myq_ref, out_hbm.at[pl.ds(base * 16, per * 16)])


def _tc_count_kernel(hi_ref, mid_ref, ticol_ref, srow_ref, qout_ref,
                     q_ref, oh_ref, *, blk, nblk, ioff):
    j = pl.program_id(0)
    i = pl.program_id(1)
    ig = i + ioff
    cp = hi_ref.shape[1]
    isl = pl.ds(i * blk, blk)

    @pl.when(j == 0)
    def _build_oh():
        oh_ref[isl, :] = (lax.broadcasted_iota(jnp.int32, (blk, cp), 1)
                          == ticol_ref[...]).astype(jnp.bfloat16)

    ohb = oh_ref[isl, :]
    dn = (((1,), (1,)), ((), ()))
    g_mat = (lax.dot_general(hi_ref[...], ohb, dn,
                             preferred_element_type=jnp.float32)
             + lax.dot_general(mid_ref[...], ohb, dn,
                               preferred_element_type=jnp.float32))
    s_i = srow_ref[...]

    def _store(q_part):
        qp = jnp.sum(q_part.astype(jnp.float32), axis=0, keepdims=True)

        @pl.when(j == 0)
        def _():
            q_ref[:, isl] = qp

        @pl.when(j != 0)
        def _():
            q_ref[:, isl] = q_ref[:, isl] + qp

    @pl.when(j < ig)
    def _before():
        _store(g_mat >= s_i)

    @pl.when(j > ig)
    def _after():
        _store(g_mat > s_i)

    @pl.when(j == ig)
    def _diag():
        jio = lax.broadcasted_iota(jnp.int32, (blk, blk), 0)
        iio = lax.broadcasted_iota(jnp.int32, (blk, blk), 1)
        _store(((g_mat > s_i) | ((g_mat == s_i) & (jio < iio)))
               & (jio != iio))

    @pl.when(j == nblk - 1)
    def _finish():
        qout_ref[...] = jnp.transpose(q_ref[:, isl])


def _final_kernel(qpart_ref, qtc_ref, ticol_ref, scol_ref, srow_ref,
                  trow_ref, hist_ref, acc_ref, *, blk, batch, num_classes,
                  nsc_blocks):
    i = pl.program_id(0)
    i0 = i * blk
    q_sc = jnp.sum(qpart_ref[...], axis=1, keepdims=True)
    q = jnp.where(i < nsc_blocks, q_sc, qtc_ref[...])
    s_i = scol_ref[...]
    t_i = ticol_ref[...]
    s_all = srow_ref[...]
    t_all = trow_ref[...]
    tmatch = t_i == t_all
    jv = lax.broadcasted_iota(jnp.int32, (blk, batch), 1)
    iv = i0 + lax.broadcasted_iota(jnp.int32, (blk, batch), 0)
    before = jv < iv
    mk = tmatch & ((s_all > s_i) | ((s_all == s_i) & before))
    k = jnp.sum(mk.astype(jnp.float32), axis=1, keepdims=True)
    p_i = jnp.sum(tmatch.astype(jnp.float32), axis=1, keepdims=True)
    bf = jnp.float32(batch)

    def g_of(x, p):
        return jnp.exp(_GAMMA * jnp.log(1.0 - x / (p + _EPS)))

    gk = g_of(k, p_i)
    gk1 = g_of(k + 1.0, p_i)
    t_q = q * (q + 1.0) * 0.5
    contrib = (q * (gk1 * (k + 1.0) - gk * k)
               + t_q * (gk - gk1)) / (bf - p_i + _EPS)
    focal = jnp.sum((1.0 - s_i) ** 2 * jnp.log(s_i))
    val = jnp.reshape(
        ((1.0 - _ALPHA) * _BETA * jnp.sum(contrib)
         + _ALPHA * _DELTA * focal) / bf, (1, 1))

    @pl.when(i == 0)
    def _init():
        hist = hist_ref[...]
        cp = hist.shape[1]
        mask = (lax.broadcasted_iota(jnp.int32, (1, cp), 1) < num_classes)
        gp = jnp.exp(_GAMMA * jnp.log(_EPS / (hist + _EPS)))
        tb = bf * (bf + 1.0) * 0.5
        ct = jnp.where(mask, gp * (tb - hist * bf) / (bf - hist + _EPS), 0.0)
        acc_ref[...] = val + jnp.reshape(
            jnp.sum(ct) * ((1.0 - _ALPHA) * _BETA / bf), (1, 1))

    @pl.when(i != 0)
    def _acc():
        acc_ref[...] = acc_ref[...] + val


@jax.jit
def kernel(logits, targets):
    b, c = logits.shape
    cp = 1024
    rb = 512
    blk = 1024
    fblk = 512
    sc_samples = 2048
    nworkers = 32
    per = sc_samples // nworkers
    ioff = sc_samples // blk
    nblk = b // blk
    logits_p = jnp.pad(logits, ((0, 0), (0, cp - c)),
                       constant_values=-1e30)
    t32 = targets.astype(jnp.int32)
    t_col = t32.reshape(b, 1)
    t_row = t32.reshape(1, b)

    s_hi, s_mid, scores_t, s_col, hist = pl.pallas_call(
        _stage1_kernel,
        grid=(b // rb,),
        in_specs=[
            pl.BlockSpec((rb, cp), lambda i: (i, 0)),
            pl.BlockSpec((rb, 1), lambda i: (i, 0)),
        ],
        out_specs=[
            pl.BlockSpec((rb, cp), lambda i: (i, 0)),
            pl.BlockSpec((rb, cp), lambda i: (i, 0)),
            pl.BlockSpec((cp, rb), lambda i: (0, i)),
            pl.BlockSpec((rb, 1), lambda i: (i, 0)),
            pl.BlockSpec((1, cp), lambda i: (0, 0)),
        ],
        out_shape=[
            jax.ShapeDtypeStruct((b, cp), jnp.bfloat16),
            jax.ShapeDtypeStruct((b, cp), jnp.bfloat16),
            jax.ShapeDtypeStruct((cp, b), jnp.float32),
            jax.ShapeDtypeStruct((b, 1), jnp.float32),
            jax.ShapeDtypeStruct((1, cp), jnp.float32),
        ],
    )(logits_p, t_col)

    s_flat = s_col.reshape(b)
    s_row = s_col.reshape(1, b)

    mesh = plsc.VectorSubcoreMesh(core_axis_name="c", subcore_axis_name="s")
    qpart = pl.kernel(
        functools.partial(_sc_count_kernel, batch=b, per=per, unroll=8),
        mesh=mesh,
        out_type=jax.ShapeDtypeStruct((sc_samples * 16,), jnp.float32),
        scratch_types=[
            pltpu.VMEM((per,), jnp.float32),
            pltpu.VMEM((per,), jnp.int32),
            pltpu.VMEM((8, b), jnp.float32),
            pltpu.VMEM((8, b), jnp.float32),
            pltpu.VMEM((per * 16,), jnp.float32),
            pltpu.VMEM((16,), jnp.float32),
            pltpu.SemaphoreType.DMA,
            pltpu.SemaphoreType.DMA,
        ],
    )(scores_t, s_flat, t32)
    qpart = jnp.pad(qpart.reshape(sc_samples, 16),
                    ((0, b - sc_samples), (0, 0)))

    q_tc = pl.pallas_call(
        functools.partial(_tc_count_kernel, blk=blk, nblk=nblk, ioff=ioff),
        grid=(nblk, (b - sc_samples) // blk),
        in_specs=[
            pl.BlockSpec((blk, cp), lambda j, i: (j, 0)),
            pl.BlockSpec((blk, cp), lambda j, i: (j, 0)),
            pl.BlockSpec((blk, 1), lambda j, i, io=ioff: (i + io, 0)),
            pl.BlockSpec((1, blk), lambda j, i, io=ioff: (0, i + io)),
        ],
        out_specs=pl.BlockSpec((blk, 1), lambda j, i, io=ioff: (i + io, 0)),
        out_shape=jax.ShapeDtypeStruct((b, 1), jnp.float32),
        scratch_shapes=[
            pltpu.VMEM((1, b - sc_samples), jnp.float32),
            pltpu.VMEM((b - sc_samples, cp), jnp.bfloat16),
        ],
    )(s_hi, s_mid, t_col, s_row)

    acc = pl.pallas_call(
        functools.partial(_final_kernel, blk=fblk, batch=b, num_classes=c,
                          nsc_blocks=sc_samples // fblk),
        grid=(b // fblk,),
        in_specs=[
            pl.BlockSpec((fblk, 16), lambda i: (i, 0)),
            pl.BlockSpec((fblk, 1), lambda i: (i, 0)),
            pl.BlockSpec((fblk, 1), lambda i: (i, 0)),
            pl.BlockSpec((fblk, 1), lambda i: (i, 0)),
            pl.BlockSpec((1, b), lambda i: (0, 0)),
            pl.BlockSpec((1, b), lambda i: (0, 0)),
            pl.BlockSpec((1, cp), lambda i: (0, 0)),
        ],
        out_specs=pl.BlockSpec((1, 1), lambda i: (0, 0)),
        out_shape=jax.ShapeDtypeStruct((1, 1), jnp.float32),
    )(qpart, q_tc, t_col, s_col, s_row, t_row, hist)

    return acc[0, 0]
